# baseline (speedup 1.0000x reference)
import functools

import jax
import jax.numpy as jnp
from jax import lax
from jax.experimental import pallas as pl
from jax.experimental.pallas import tpu as pltpu
from jax.experimental.pallas import tpu_sc as plsc


def _gather_kernel(B, D, NC, NW, CHUNK):
    n_chunks_per_w = B // (NW * CHUNK)
    b_per_w = B // NW
    mesh = plsc.VectorSubcoreMesh(core_axis_name="c", subcore_axis_name="s")

    @functools.partial(
        pl.kernel,
        mesh=mesh,
        out_type=jax.ShapeDtypeStruct((B, D), jnp.float32),
        scratch_types=[
            pltpu.VMEM((CHUNK,), jnp.int32),
            pltpu.VMEM((CHUNK, D), jnp.float32),
            pltpu.SemaphoreType.DMA,
        ],
        compiler_params=pltpu.CompilerParams(use_tc_tiling_on_sc=False),
    )
    def k(table_hbm, idx_hbm, out_hbm, idx_v, rows_v, sem):
        wid = lax.axis_index("s") * NC + lax.axis_index("c")
        base = wid * b_per_w

        def body(i, carry):
            pltpu.sync_copy(idx_hbm.at[wid, i], idx_v)
            pltpu.async_copy(table_hbm.at[idx_v], rows_v, sem).wait()
            pltpu.sync_copy(rows_v, out_hbm.at[pl.ds(base + i * CHUNK, CHUNK)])
            return carry

        lax.fori_loop(0, n_chunks_per_w, body, 0)

    return k


def kernel(x, weights):
    Bdim, S = x.shape
    V, D = weights.shape
    B = Bdim * S
    info = plsc.get_sparse_core_info()
    NC, NS = info.num_cores, info.num_subcores
    NW = NC * NS
    CHUNK = 800
    flat_idx = x.reshape(NW, B // (NW * CHUNK), CHUNK).astype(jnp.int32)
    k = _gather_kernel(B, D, NC, NW, CHUNK)
    out = k(weights, flat_idx)
    return out.reshape(Bdim, S, D)


# speedup vs baseline: 1.5004x; 1.5004x over previous
import functools

import jax
import jax.numpy as jnp
from jax import lax
from jax.experimental import pallas as pl
from jax.experimental.pallas import tpu as pltpu
from jax.experimental.pallas import tpu_sc as plsc


def _gather_kernel(B, D, V, NC, NW, CHUNK):
    n_chunks = B // (NW * CHUNK)
    b_per_w = B // NW
    mesh = plsc.VectorSubcoreMesh(core_axis_name="c", subcore_axis_name="s")

    @functools.partial(
        pl.kernel,
        mesh=mesh,
        out_type=jax.ShapeDtypeStruct((B, D), jnp.float32),
        scratch_types=[
            pltpu.VMEM_SHARED((V, D), jnp.float32),
            pltpu.VMEM((b_per_w,), jnp.int32),
            pltpu.VMEM((CHUNK, D), jnp.float32),
            pltpu.VMEM((CHUNK, D), jnp.float32),
            pltpu.SemaphoreType.DMA,
            pltpu.SemaphoreType.DMA,
            pltpu.SemaphoreType.DMA,
            pltpu.SemaphoreType.DMA,
        ],
        compiler_params=pltpu.CompilerParams(use_tc_tiling_on_sc=False),
    )
    def k(table_hbm, idx_hbm, out_hbm, table_sp, idx_v, rows0, rows1,
          gsem0, gsem1, wsem0, wsem1):
        sid = lax.axis_index("s")
        wid = sid * NC + lax.axis_index("c")
        base = wid * b_per_w

        @pl.when(sid == 0)
        def _():
            pltpu.sync_copy(table_hbm, table_sp)

        plsc.subcore_barrier()
        pltpu.sync_copy(idx_hbm.at[wid], idx_v)

        bufs = (rows0, rows1)
        gsems = (gsem0, gsem1)
        wsems = (wsem0, wsem1)

        def start_gather(i, b):
            return pltpu.async_copy(
                table_sp.at[idx_v.at[pl.ds(i * CHUNK, CHUNK)]], bufs[b],
                gsems[b])

        gh = [None, None]
        wh = [None, None]
        gh[0] = start_gather(0, 0)
        for i in range(n_chunks):
            b = i % 2
            if i + 1 < n_chunks:
                if wh[1 - b] is not None:
                    wh[1 - b].wait()
                gh[1 - b] = start_gather(i + 1, 1 - b)
            gh[b].wait()
            wh[b] = pltpu.async_copy(
                bufs[b], out_hbm.at[pl.ds(base + i * CHUNK, CHUNK)], wsems[b])
        for b in range(2):
            if wh[b] is not None:
                wh[b].wait()

    return k


def kernel(x, weights):
    Bdim, S = x.shape
    V, D = weights.shape
    B = Bdim * S
    info = plsc.get_sparse_core_info()
    NC, NS = info.num_cores, info.num_subcores
    NW = NC * NS
    CHUNK = 800
    flat_idx = x.reshape(NW, B // NW).astype(jnp.int32)
    k = _gather_kernel(B, D, V, NC, NW, CHUNK)
    out = k(weights, flat_idx)
    return out.reshape(Bdim, S, D)
